# baseline (speedup 1.0000x reference)
import functools

import jax
import jax.numpy as jnp
from jax import lax
from jax.experimental import pallas as pl
from jax.experimental.pallas import tpu as pltpu
from jax.experimental.pallas import tpu_sc as plsc

_NC = 2
_NS = 16
_NW = _NC * _NS
_ROW = 128
_L = 16


def _edge_proj(ea, w1c):
    E = ea.shape[0]
    blk = 4096
    grid = pl.cdiv(E, blk)

    def body(ea_ref, w_ref, o_ref):
        o_ref[...] = jnp.dot(ea_ref[...], w_ref[...],
                             preferred_element_type=jnp.float32)

    return pl.pallas_call(
        body,
        grid=(grid,),
        in_specs=[
            pl.BlockSpec((blk, ea.shape[1]), lambda i: (i, 0)),
            pl.BlockSpec(w1c.shape, lambda i: (0, 0)),
        ],
        out_specs=pl.BlockSpec((blk, w1c.shape[1]), lambda i: (i, 0)),
        out_shape=jax.ShapeDtypeStruct((E, w1c.shape[1]), jnp.float32),
    )(ea, w1c)


def _node_proj(x_s, x_t, w1a, w1b):
    n = x_s.shape[0]
    blk = 5000
    grid = pl.cdiv(n, blk)

    def body(xs_ref, xt_ref, wa_ref, wb_ref, a_ref, b_ref):
        a_ref[...] = jnp.dot(xs_ref[...], wa_ref[...],
                             preferred_element_type=jnp.float32)
        b_ref[...] = jnp.dot(xt_ref[...], wb_ref[...],
                             preferred_element_type=jnp.float32)

    return pl.pallas_call(
        body,
        grid=(grid,),
        in_specs=[
            pl.BlockSpec((blk, x_s.shape[1]), lambda i: (i, 0)),
            pl.BlockSpec((blk, x_t.shape[1]), lambda i: (i, 0)),
            pl.BlockSpec(w1a.shape, lambda i: (0, 0)),
            pl.BlockSpec(w1b.shape, lambda i: (0, 0)),
        ],
        out_specs=[
            pl.BlockSpec((blk, w1a.shape[1]), lambda i: (i, 0)),
            pl.BlockSpec((blk, w1b.shape[1]), lambda i: (i, 0)),
        ],
        out_shape=[
            jax.ShapeDtypeStruct((n, w1a.shape[1]), jnp.float32),
            jax.ShapeDtypeStruct((n, w1b.shape[1]), jnp.float32),
        ],
    )(x_s, x_t, w1a, w1b)


def _u_proj(u, w1u, b1):
    def body(u_ref, w_ref, b_ref, o_ref):
        o_ref[...] = (jnp.dot(u_ref[...], w_ref[...],
                              preferred_element_type=jnp.float32)
                      + b_ref[...])

    return pl.pallas_call(
        body,
        out_shape=jax.ShapeDtypeStruct((u.shape[0], w1u.shape[1]),
                                       jnp.float32),
    )(u, w1u, b1.reshape(1, -1))


def _sc_edge_mlp(eidx, be, c, a, bt, uq, wpack, rows):
    f_out = 5
    groups = _ROW // _L
    mesh = plsc.VectorSubcoreMesh(core_axis_name="c", subcore_axis_name="s")

    @functools.partial(
        pl.kernel,
        mesh=mesh,
        compiler_params=pltpu.CompilerParams(
            needs_layout_passes=False, use_tc_tiling_on_sc=False),
        out_type=jax.ShapeDtypeStruct((rows * _ROW, f_out), jnp.float32),
        scratch_types=[
            pltpu.VMEM((_ROW,), jnp.int32),
            pltpu.VMEM((_ROW,), jnp.int32),
            pltpu.VMEM((_ROW,), jnp.int32),
            pltpu.VMEM((_ROW, f_out), jnp.float32),
            pltpu.VMEM((_ROW, 8), jnp.float32),
            pltpu.VMEM((_ROW, 8), jnp.float32),
            pltpu.VMEM((uq.shape[0], f_out), jnp.float32),
            pltpu.VMEM((_ROW, f_out), jnp.float32),
            pltpu.VMEM((48,), jnp.float32),
            pltpu.SemaphoreType.DMA,
            pltpu.SemaphoreType.DMA,
            pltpu.SemaphoreType.DMA,
            pltpu.SemaphoreType.DMA,
            pltpu.SemaphoreType.DMA,
            pltpu.SemaphoreType.DMA,
        ],
    )
    def body(eidx_hbm, be_hbm, c_hbm, a_hbm, bt_hbm, uq_hbm, w_hbm, out_hbm,
             sidx, tidx, bidx, cv, av, bv, uqv, ov, wv,
             sem_s, sem_t, sem_b, sem_c, sem_a, sem_bt):
        wid = lax.axis_index("s") * _NC + lax.axis_index("c")
        base = (rows * wid) // _NW
        end = (rows * (wid + 1)) // _NW

        pltpu.sync_copy(uq_hbm, uqv)
        pltpu.sync_copy(w_hbm, wv)
        w2 = [[plsc.load_gather(wv, [jnp.full((_L,), 1 + 5 * k + j, jnp.int32)])
               for j in range(f_out)] for k in range(f_out)]
        b2 = [plsc.load_gather(wv, [jnp.full((_L,), 26 + j, jnp.int32)])
              for j in range(f_out)]

        def row_body(r, carry):
            e0 = (base + r) * _ROW
            ds = pltpu.async_copy(eidx_hbm.at[0, pl.ds(e0, _ROW)], sidx, sem_s)
            dt = pltpu.async_copy(eidx_hbm.at[1, pl.ds(e0, _ROW)], tidx, sem_t)
            db = pltpu.async_copy(be_hbm.at[pl.ds(e0, _ROW)], bidx, sem_b)
            dc = pltpu.async_copy(c_hbm.at[pl.ds(e0, _ROW)], cv, sem_c)
            ds.wait()
            dt.wait()
            ga = pltpu.async_copy(a_hbm.at[sidx], av, sem_a)
            gb = pltpu.async_copy(bt_hbm.at[tidx], bv, sem_bt)
            db.wait()
            dc.wait()
            ga.wait()
            gb.wait()

            for g in range(groups):
                ids = jnp.arange(_L, dtype=jnp.int32) + (_L * g)
                bvec = bidx[pl.ds(_L * g, _L)]
                h = []
                for j in range(f_out):
                    jv = jnp.full((_L,), j, jnp.int32)
                    aj = plsc.load_gather(av, [ids, jv])
                    bj = plsc.load_gather(bv, [ids, jv])
                    cj = plsc.load_gather(cv, [ids, jv])
                    uj = plsc.load_gather(uqv, [bvec, jv])
                    h.append((aj + bj) + (cj + uj))
                h = [jnp.maximum(x, 0.1 * x) for x in h]
                for j in range(f_out):
                    y = h[0] * w2[0][j]
                    for k in range(1, f_out):
                        y = y + h[k] * w2[k][j]
                    y = y + b2[j]
                    plsc.store_scatter(
                        ov, [ids, jnp.full((_L,), j, jnp.int32)], y)

            pltpu.sync_copy(ov, out_hbm.at[pl.ds(e0, _ROW)])
            return carry

        lax.fori_loop(0, end - base, row_body, 0)

    return body(eidx, be, c, a, bt, uq, wpack)


def kernel(x_s, x_t, edge_index, edge_attr, u, batch_e, W1, b1, W2, b2):
    n = x_s.shape[0]
    e = edge_index.shape[1]
    f_xs = x_s.shape[1]
    f_xt = x_t.shape[1]
    f_e = edge_attr.shape[1]

    w1a = jnp.pad(W1[:f_xs], ((0, 0), (0, 3)))
    w1b = jnp.pad(W1[f_xs:f_xs + f_xt], ((0, 0), (0, 3)))
    w1c = W1[f_xs + f_xt:f_xs + f_xt + f_e]
    w1u = W1[f_xs + f_xt + f_e:]

    a, bt = _node_proj(x_s, x_t, w1a, w1b)
    c = _edge_proj(edge_attr, w1c)
    uq = _u_proj(u, w1u, b1)

    rows = e // _ROW
    wpack = jnp.concatenate(
        [jnp.zeros((1,), jnp.float32), W2.reshape(-1), b2,
         jnp.zeros((17,), jnp.float32)])

    return _sc_edge_mlp(edge_index, batch_e, c, a, bt, uq, wpack, rows)


# speedup vs baseline: 1.3123x; 1.3123x over previous
import functools

import jax
import jax.numpy as jnp
from jax import lax
from jax.experimental import pallas as pl
from jax.experimental.pallas import tpu as pltpu
from jax.experimental.pallas import tpu_sc as plsc

_NC = 2
_NS = 16
_NW = _NC * _NS
_ROW = 128
_L = 16


def _node_proj(x_s, x_t, w1a, w1b):
    n = x_s.shape[0]
    blk = 5000
    grid = pl.cdiv(n, blk)

    def body(xs_ref, xt_ref, wa_ref, wb_ref, a_ref, b_ref):
        a_ref[...] = jnp.dot(xs_ref[...], wa_ref[...],
                             preferred_element_type=jnp.float32)
        b_ref[...] = jnp.dot(xt_ref[...], wb_ref[...],
                             preferred_element_type=jnp.float32)

    return pl.pallas_call(
        body,
        grid=(grid,),
        in_specs=[
            pl.BlockSpec((blk, x_s.shape[1]), lambda i: (i, 0)),
            pl.BlockSpec((blk, x_t.shape[1]), lambda i: (i, 0)),
            pl.BlockSpec(w1a.shape, lambda i: (0, 0)),
            pl.BlockSpec(w1b.shape, lambda i: (0, 0)),
        ],
        out_specs=[
            pl.BlockSpec((blk, w1a.shape[1]), lambda i: (i, 0)),
            pl.BlockSpec((blk, w1b.shape[1]), lambda i: (i, 0)),
        ],
        out_shape=[
            jax.ShapeDtypeStruct((n, w1a.shape[1]), jnp.float32),
            jax.ShapeDtypeStruct((n, w1b.shape[1]), jnp.float32),
        ],
    )(x_s, x_t, w1a, w1b)


def _u_proj(u, w1u, b1):
    def body(u_ref, w_ref, b_ref, o_ref):
        o_ref[...] = (jnp.dot(u_ref[...], w_ref[...],
                              preferred_element_type=jnp.float32)
                      + b_ref[...])

    return pl.pallas_call(
        body,
        out_shape=jax.ShapeDtypeStruct((u.shape[0], w1u.shape[1]),
                                       jnp.float32),
    )(u, w1u, b1.reshape(1, -1))


def _sc_edge_mlp(eidx, be, ea, a, bt, uq, wpack, rows):
    f_out = 5
    f_e = 10
    groups = _ROW // _L
    mesh = plsc.VectorSubcoreMesh(core_axis_name="c", subcore_axis_name="s")

    @functools.partial(
        pl.kernel,
        mesh=mesh,
        compiler_params=pltpu.CompilerParams(
            needs_layout_passes=False, use_tc_tiling_on_sc=False),
        out_type=jax.ShapeDtypeStruct((rows * _ROW, f_out), jnp.float32),
        scratch_types=[
            pltpu.VMEM((_ROW,), jnp.int32),
            pltpu.VMEM((_ROW,), jnp.int32),
            pltpu.VMEM((_ROW,), jnp.int32),
            pltpu.VMEM((_ROW, f_e), jnp.float32),
            pltpu.VMEM((_ROW, 8), jnp.float32),
            pltpu.VMEM((_ROW, 8), jnp.float32),
            pltpu.VMEM((uq.shape[0], f_out), jnp.float32),
            pltpu.VMEM((_ROW, f_out), jnp.float32),
            pltpu.VMEM((96,), jnp.float32),
            pltpu.SemaphoreType.DMA,
            pltpu.SemaphoreType.DMA,
            pltpu.SemaphoreType.DMA,
            pltpu.SemaphoreType.DMA,
            pltpu.SemaphoreType.DMA,
            pltpu.SemaphoreType.DMA,
        ],
    )
    def body(eidx_hbm, be_hbm, ea_hbm, a_hbm, bt_hbm, uq_hbm, w_hbm, out_hbm,
             sidx, tidx, bidx, eav, av, bv, uqv, ov, wv,
             sem_s, sem_t, sem_b, sem_e, sem_a, sem_bt):
        wid = lax.axis_index("s") * _NC + lax.axis_index("c")
        base = (rows * wid) // _NW
        end = (rows * (wid + 1)) // _NW

        pltpu.sync_copy(uq_hbm, uqv)
        pltpu.sync_copy(w_hbm, wv)
        wvecs = [wv[pl.ds(16 * i, 16)] for i in range(5)]
        wsc = [wvecs[i // 16][i % 16] for i in range(80)]
        w1c = [[wsc[f_out * k + j] for j in range(f_out)] for k in range(f_e)]
        w2 = [[wsc[50 + f_out * k + j] for j in range(f_out)]
              for k in range(f_out)]
        b2 = [wsc[75 + j] for j in range(f_out)]

        def row_body(r, carry):
            e0 = (base + r) * _ROW
            ds = pltpu.async_copy(eidx_hbm.at[0, pl.ds(e0, _ROW)], sidx, sem_s)
            dt = pltpu.async_copy(eidx_hbm.at[1, pl.ds(e0, _ROW)], tidx, sem_t)
            db = pltpu.async_copy(be_hbm.at[pl.ds(e0, _ROW)], bidx, sem_b)
            de = pltpu.async_copy(ea_hbm.at[pl.ds(e0, _ROW)], eav, sem_e)
            ds.wait()
            dt.wait()
            ga = pltpu.async_copy(a_hbm.at[sidx], av, sem_a)
            gb = pltpu.async_copy(bt_hbm.at[tidx], bv, sem_bt)
            db.wait()
            de.wait()
            ga.wait()
            gb.wait()

            for g in range(groups):
                ids = jnp.arange(_L, dtype=jnp.int32) + (_L * g)
                bvec = bidx[pl.ds(_L * g, _L)]
                e_k = [plsc.load_gather(eav, [ids, jnp.full((_L,), k, jnp.int32)])
                       for k in range(f_e)]
                h = []
                for j in range(f_out):
                    jv = jnp.full((_L,), j, jnp.int32)
                    aj = plsc.load_gather(av, [ids, jv])
                    bj = plsc.load_gather(bv, [ids, jv])
                    uj = plsc.load_gather(uqv, [bvec, jv])
                    x = (aj + bj) + uj
                    for k in range(f_e):
                        x = x + e_k[k] * w1c[k][j]
                    h.append(x)
                h = [jnp.maximum(x, 0.1 * x) for x in h]
                for j in range(f_out):
                    y = h[0] * w2[0][j]
                    for k in range(1, f_out):
                        y = y + h[k] * w2[k][j]
                    y = y + b2[j]
                    plsc.store_scatter(
                        ov, [ids, jnp.full((_L,), j, jnp.int32)], y)

            pltpu.sync_copy(ov, out_hbm.at[pl.ds(e0, _ROW)])
            return carry

        lax.fori_loop(0, end - base, row_body, 0)

    return body(eidx, be, ea, a, bt, uq, wpack)


def kernel(x_s, x_t, edge_index, edge_attr, u, batch_e, W1, b1, W2, b2):
    e = edge_index.shape[1]
    f_xs = x_s.shape[1]
    f_xt = x_t.shape[1]
    f_e = edge_attr.shape[1]

    w1a = jnp.pad(W1[:f_xs], ((0, 0), (0, 3)))
    w1b = jnp.pad(W1[f_xs:f_xs + f_xt], ((0, 0), (0, 3)))
    w1c = W1[f_xs + f_xt:f_xs + f_xt + f_e]
    w1u = W1[f_xs + f_xt + f_e:]

    a, bt = _node_proj(x_s, x_t, w1a, w1b)
    uq = _u_proj(u, w1u, b1)

    rows = e // _ROW
    wpack = jnp.concatenate(
        [w1c.reshape(-1), W2.reshape(-1), b2,
         jnp.zeros((16,), jnp.float32)])

    return _sc_edge_mlp(edge_index, batch_e, edge_attr, a, bt, uq, wpack,
                        rows)


# speedup vs baseline: 1.3373x; 1.0191x over previous
import functools

import jax
import jax.numpy as jnp
from jax import lax
from jax.experimental import pallas as pl
from jax.experimental.pallas import tpu as pltpu
from jax.experimental.pallas import tpu_sc as plsc

_NC = 2
_NS = 16
_NW = _NC * _NS
_ROW = 128
_L = 16


def _node_proj(x_s, x_t, w1a, w1b):
    n = x_s.shape[0]
    blk = 5000
    grid = pl.cdiv(n, blk)

    def body(xs_ref, xt_ref, wa_ref, wb_ref, a_ref, b_ref):
        a_ref[...] = jnp.dot(xs_ref[...], wa_ref[...],
                             preferred_element_type=jnp.float32)
        b_ref[...] = jnp.dot(xt_ref[...], wb_ref[...],
                             preferred_element_type=jnp.float32)

    return pl.pallas_call(
        body,
        grid=(grid,),
        in_specs=[
            pl.BlockSpec((blk, x_s.shape[1]), lambda i: (i, 0)),
            pl.BlockSpec((blk, x_t.shape[1]), lambda i: (i, 0)),
            pl.BlockSpec(w1a.shape, lambda i: (0, 0)),
            pl.BlockSpec(w1b.shape, lambda i: (0, 0)),
        ],
        out_specs=[
            pl.BlockSpec((blk, w1a.shape[1]), lambda i: (i, 0)),
            pl.BlockSpec((blk, w1b.shape[1]), lambda i: (i, 0)),
        ],
        out_shape=[
            jax.ShapeDtypeStruct((n, w1a.shape[1]), jnp.float32),
            jax.ShapeDtypeStruct((n, w1b.shape[1]), jnp.float32),
        ],
    )(x_s, x_t, w1a, w1b)


def _u_proj(u, w1u, b1):
    def body(u_ref, w_ref, b_ref, o_ref):
        o_ref[...] = (jnp.dot(u_ref[...], w_ref[...],
                              preferred_element_type=jnp.float32)
                      + b_ref[...])

    return pl.pallas_call(
        body,
        out_shape=jax.ShapeDtypeStruct((u.shape[0], w1u.shape[1]),
                                       jnp.float32),
    )(u, w1u, b1.reshape(1, -1))


def _sc_edge_mlp(eidx, be, ea, a, bt, uq, wpack, rows):
    f_out = 5
    f_e = 10
    groups = _ROW // _L
    mesh = plsc.VectorSubcoreMesh(core_axis_name="c", subcore_axis_name="s")

    @functools.partial(
        pl.kernel,
        mesh=mesh,
        compiler_params=pltpu.CompilerParams(
            needs_layout_passes=False, use_tc_tiling_on_sc=False),
        out_type=jax.ShapeDtypeStruct((rows * _ROW * f_out,), jnp.float32),
        scratch_types=[
            pltpu.VMEM((_ROW,), jnp.int32),
            pltpu.VMEM((_ROW,), jnp.int32),
            pltpu.VMEM((_ROW,), jnp.int32),
            pltpu.VMEM((_ROW * f_e,), jnp.float32),
            pltpu.VMEM((_ROW, 8), jnp.float32),
            pltpu.VMEM((_ROW, 8), jnp.float32),
            pltpu.VMEM((uq.shape[0], f_out), jnp.float32),
            pltpu.VMEM((_ROW * f_out,), jnp.float32),
            pltpu.VMEM((96,), jnp.float32),
            pltpu.SemaphoreType.DMA,
            pltpu.SemaphoreType.DMA,
            pltpu.SemaphoreType.DMA,
            pltpu.SemaphoreType.DMA,
            pltpu.SemaphoreType.DMA,
            pltpu.SemaphoreType.DMA,
        ],
    )
    def body(eidx_hbm, be_hbm, ea_hbm, a_hbm, bt_hbm, uq_hbm, w_hbm, out_hbm,
             sidx, tidx, bidx, eav, av, bv, uqv, ov, wv,
             sem_s, sem_t, sem_b, sem_e, sem_a, sem_bt):
        wid = lax.axis_index("s") * _NC + lax.axis_index("c")
        base = (rows * wid) // _NW
        end = (rows * (wid + 1)) // _NW

        pltpu.sync_copy(uq_hbm, uqv)
        pltpu.sync_copy(w_hbm, wv)
        wvecs = [wv[pl.ds(16 * i, 16)] for i in range(5)]
        wsc = [wvecs[i // 16][i % 16] for i in range(80)]
        w1c = [[wsc[f_out * k + j] for j in range(f_out)] for k in range(f_e)]
        w2 = [[wsc[50 + f_out * k + j] for j in range(f_out)]
              for k in range(f_out)]
        b2 = [wsc[75 + j] for j in range(f_out)]

        n_e = rows * _ROW

        def row_body(r, carry):
            e0 = (base + r) * _ROW
            ds = pltpu.async_copy(eidx_hbm.at[pl.ds(e0, _ROW)], sidx, sem_s)
            dt = pltpu.async_copy(eidx_hbm.at[pl.ds(n_e + e0, _ROW)], tidx,
                                  sem_t)
            db = pltpu.async_copy(be_hbm.at[pl.ds(e0, _ROW)], bidx, sem_b)
            de = pltpu.async_copy(ea_hbm.at[pl.ds(e0 * f_e, _ROW * f_e)], eav,
                                  sem_e)
            ds.wait()
            dt.wait()
            ga = pltpu.async_copy(a_hbm.at[sidx], av, sem_a)
            gb = pltpu.async_copy(bt_hbm.at[tidx], bv, sem_bt)
            db.wait()
            de.wait()
            ga.wait()
            gb.wait()

            for g in range(groups):
                ids = jnp.arange(_L, dtype=jnp.int32) + (_L * g)
                iota_fe = jnp.arange(_L, dtype=jnp.int32) * f_e
                bvec = bidx[pl.ds(_L * g, _L)]
                e_k = [plsc.load_gather(eav, [iota_fe + (_L * f_e * g + k)])
                       for k in range(f_e)]
                h = []
                for j in range(f_out):
                    jv = jnp.full((_L,), j, jnp.int32)
                    aj = plsc.load_gather(av, [ids, jv])
                    bj = plsc.load_gather(bv, [ids, jv])
                    uj = plsc.load_gather(uqv, [bvec, jv])
                    x = (aj + bj) + uj
                    for k in range(f_e):
                        x = x + e_k[k] * w1c[k][j]
                    h.append(x)
                h = [jnp.maximum(x, 0.1 * x) for x in h]
                iota_fo = jnp.arange(_L, dtype=jnp.int32) * f_out
                for j in range(f_out):
                    y = h[0] * w2[0][j]
                    for k in range(1, f_out):
                        y = y + h[k] * w2[k][j]
                    y = y + b2[j]
                    plsc.store_scatter(
                        ov, [iota_fo + (_L * f_out * g + j)], y)

            pltpu.sync_copy(ov, out_hbm.at[pl.ds(e0 * f_out, _ROW * f_out)])
            return carry

        lax.fori_loop(0, end - base, row_body, 0)

    return body(eidx, be, ea, a, bt, uq, wpack)


def kernel(x_s, x_t, edge_index, edge_attr, u, batch_e, W1, b1, W2, b2):
    e = edge_index.shape[1]
    f_xs = x_s.shape[1]
    f_xt = x_t.shape[1]
    f_e = edge_attr.shape[1]

    w1a = jnp.pad(W1[:f_xs], ((0, 0), (0, 3)))
    w1b = jnp.pad(W1[f_xs:f_xs + f_xt], ((0, 0), (0, 3)))
    w1c = W1[f_xs + f_xt:f_xs + f_xt + f_e]
    w1u = W1[f_xs + f_xt + f_e:]

    a, bt = _node_proj(x_s, x_t, w1a, w1b)
    uq = _u_proj(u, w1u, b1)

    rows = e // _ROW
    wpack = jnp.concatenate(
        [w1c.reshape(-1), W2.reshape(-1), b2,
         jnp.zeros((16,), jnp.float32)])

    out = _sc_edge_mlp(edge_index.reshape(-1), batch_e,
                       edge_attr.reshape(-1), a, bt, uq, wpack, rows)
    return out.reshape(e, 5)
